# baseline (speedup 1.0000x reference)
import functools

import jax
import jax.numpy as jnp
from jax import lax
from jax.experimental import pallas as pl
from jax.experimental.pallas import tpu as pltpu
from jax.experimental.pallas import tpu_sc as plsc

L = 16


@functools.lru_cache(maxsize=None)
def _build(B, V, D, eps):
    info = plsc.get_sparse_core_info()
    NC, NS = info.num_cores, info.num_subcores
    NW = NC * NS
    assert B % NW == 0
    per_w = B // NW
    C = 64
    while per_w % C:
        C //= 2
    n_chunks = per_w // C
    n_sl = D // L
    inv_d = 1.0 / D

    gather_dnums = lax.GatherDimensionNumbers(
        offset_dims=(), collapsed_slice_dims=(0,), start_index_map=(0,))

    def _shuffle(v, perm):
        return lax.gather(v, perm[:, None], gather_dnums, slice_sizes=(1,),
                          mode=lax.GatherScatterMode.PROMISE_IN_BOUNDS)

    mesh = plsc.VectorSubcoreMesh(core_axis_name="c", subcore_axis_name="s")

    @functools.partial(
        pl.kernel,
        mesh=mesh,
        out_type=jax.ShapeDtypeStruct((B, D), jnp.float32),
        scratch_types=[
            pltpu.VMEM((C,), jnp.int32),
            pltpu.VMEM((C, D), jnp.float32),
            pltpu.VMEM((D,), jnp.float32),
            pltpu.VMEM((D,), jnp.float32),
            pltpu.VMEM((D,), jnp.float32),
            pltpu.VMEM((D,), jnp.float32),
            pltpu.SemaphoreType.DMA,
        ],
    )
    def sc_kernel(ids_hbm, table_hbm, scale_hbm, bias_hbm, lnw_hbm, lnb_hbm,
                  out_hbm, idx_v, rows_v, scale_v, bias_v, lnw_v, lnb_v, sem):
        wid = lax.axis_index("s") * NC + lax.axis_index("c")
        base = wid * per_w
        pltpu.sync_copy(scale_hbm, scale_v)
        pltpu.sync_copy(bias_hbm, bias_v)
        pltpu.sync_copy(lnw_hbm, lnw_v)
        pltpu.sync_copy(lnb_hbm, lnb_v)

        def chunk_body(g, carry):
            cbase = base + g * C
            pltpu.sync_copy(ids_hbm.at[pl.ds(cbase, C)], idx_v)
            pltpu.async_copy(table_hbm.at[idx_v], rows_v, sem).wait()

            def row_body(r, carry2):
                def p1(d, acc):
                    s, ss = acc
                    sl = pl.ds(d * L, L)
                    y = rows_v[r, sl] * scale_v[sl] + bias_v[sl]
                    rows_v[r, sl] = y
                    return (s + y, ss + y * y)

                zero = jnp.zeros((L,), jnp.float32)
                s, ss = lax.fori_loop(0, n_sl, p1, (zero, zero))
                for sh in (8, 4, 2, 1):
                    perm = lax.iota(jnp.int32, L) ^ sh
                    s = s + _shuffle(s, perm)
                    ss = ss + _shuffle(ss, perm)
                mean_v = s * inv_d
                var = ss * inv_d - mean_v * mean_v
                x = var + eps
                i = lax.bitcast_convert_type(x, jnp.int32)
                i = 0x5F3759DF - lax.shift_right_logical(i, 1)
                y0 = lax.bitcast_convert_type(i, jnp.float32)
                half_x = 0.5 * x
                for _ in range(3):
                    y0 = y0 * (1.5 - half_x * y0 * y0)
                rstd = y0

                def p2(d, c):
                    sl = pl.ds(d * L, L)
                    y = rows_v[r, sl]
                    rows_v[r, sl] = (y - mean_v) * rstd * lnw_v[sl] + lnb_v[sl]
                    return c

                return lax.fori_loop(0, n_sl, p2, carry2)

            lax.fori_loop(0, C, row_body, 0)
            pltpu.sync_copy(rows_v, out_hbm.at[pl.ds(cbase, C)])
            return carry

        lax.fori_loop(0, n_chunks, chunk_body, 0)

    return sc_kernel


def kernel(input_ids, table, scale, bias, ln_weight, ln_bias):
    bsz, seq = input_ids.shape
    v, d = table.shape
    ids = input_ids.reshape(-1).astype(jnp.int32)
    fn = _build(bsz * seq, v, d, 1e-6)
    out = fn(ids, table, scale, bias, ln_weight, ln_bias)
    return out.reshape(bsz, seq, d)
